# baseline (speedup 1.0000x reference)
import functools

import jax
import jax.numpy as jnp
from jax.experimental import pallas as pl
from jax.experimental.pallas import tpu as pltpu

TOPK = 2
E = 16
BLOCK_M = 512


def _router_kernel(x_ref, wt_ref, rw_ref, sel_ref, aux_ref, acc_ref,
                   *, nblocks, inv_total):
    i = pl.program_id(0)
    x = x_ref[...]
    wt = wt_ref[...]
    logits = jnp.dot(x, wt, preferred_element_type=jnp.float32)

    m1 = jnp.max(logits, axis=1, keepdims=True)
    i1 = jnp.argmax(logits, axis=1)
    eidx = jax.lax.broadcasted_iota(jnp.int32, logits.shape, 1)
    masked = jnp.where(eidx == i1[:, None], -jnp.inf, logits)
    m2 = jnp.max(masked, axis=1, keepdims=True)
    i2 = jnp.argmax(masked, axis=1)

    e2 = jnp.exp(m2 - m1)
    denom = 1.0 + e2
    w1 = 1.0 / denom
    w2 = e2 / denom
    rw_ref[...] = jnp.concatenate([w1, w2], axis=1)
    sel_ref[...] = jnp.concatenate([i1[:, None], i2[:, None]], axis=1)

    p = jnp.exp(logits - m1)
    p = p / jnp.sum(p, axis=1, keepdims=True)
    psum = jnp.sum(p, axis=0, keepdims=True)

    @pl.when(i == 0)
    def _():
        acc_ref[...] = jnp.zeros_like(acc_ref)

    acc_ref[...] += psum

    @pl.when(i == nblocks - 1)
    def _():
        mean_pe = acc_ref[...] * inv_total
        aux_ref[...] = E * jnp.sum(mean_pe * mean_pe, axis=(0, 1),
                                   keepdims=True)


def kernel(hidden_states, gate_weight):
    b, s, h = hidden_states.shape
    n = b * s
    x = hidden_states.reshape(n, h)
    wt = gate_weight.T
    nblocks = n // BLOCK_M

    body = functools.partial(_router_kernel, nblocks=nblocks,
                             inv_total=1.0 / n)
    rw, sel, aux = pl.pallas_call(
        body,
        grid=(nblocks,),
        in_specs=[
            pl.BlockSpec((BLOCK_M, h), lambda i: (i, 0)),
            pl.BlockSpec((h, E), lambda i: (0, 0)),
        ],
        out_specs=[
            pl.BlockSpec((BLOCK_M, TOPK), lambda i: (i, 0)),
            pl.BlockSpec((BLOCK_M, TOPK), lambda i: (i, 0)),
            pl.BlockSpec((1, 1), lambda i: (0, 0)),
        ],
        out_shape=[
            jax.ShapeDtypeStruct((n, TOPK), jnp.float32),
            jax.ShapeDtypeStruct((n, TOPK), jnp.int32),
            jax.ShapeDtypeStruct((1, 1), jnp.float32),
        ],
        scratch_shapes=[pltpu.VMEM((1, E), jnp.float32)],
    )(x, wt)

    return (rw.reshape(b, s, TOPK), sel.reshape(b, s, TOPK), aux[0, 0])


# speedup vs baseline: 1.1487x; 1.1487x over previous
import functools

import jax
import jax.numpy as jnp
from jax.experimental import pallas as pl
from jax.experimental.pallas import tpu as pltpu

TOPK = 2
E = 16
BLOCK_M = 2048
NSTREAM = 2
SUB_M = BLOCK_M // NSTREAM


def _route_block(logits, rw_ref, sel_ref):
    m1 = jnp.max(logits, axis=1, keepdims=True)
    i1 = jnp.argmax(logits, axis=1)
    eidx = jax.lax.broadcasted_iota(jnp.int32, logits.shape, 1)
    masked = jnp.where(eidx == i1[:, None], -jnp.inf, logits)
    m2 = jnp.max(masked, axis=1, keepdims=True)
    i2 = jnp.argmax(masked, axis=1)

    e2 = jnp.exp(m2 - m1)
    denom = 1.0 + e2
    rw_ref[...] = jnp.concatenate([1.0 / denom, e2 / denom], axis=1)
    sel_ref[...] = jnp.concatenate([i1[:, None], i2[:, None]], axis=1)

    p = jnp.exp(logits - m1)
    p = p / jnp.sum(p, axis=1, keepdims=True)
    return jnp.sum(p, axis=0, keepdims=True)


def _router_kernel(xa_ref, xb_ref, wt_ref, rw_ref, sel_ref, aux_ref,
                   acc_ref, *, nblocks, inv_total):
    i = pl.program_id(0)
    wt = wt_ref[...]
    la = jnp.dot(xa_ref[...], wt, preferred_element_type=jnp.float32)
    lb = jnp.dot(xb_ref[...], wt, preferred_element_type=jnp.float32)
    pa = _route_block(la, rw_ref.at[pl.ds(0, SUB_M), :],
                      sel_ref.at[pl.ds(0, SUB_M), :])
    pb = _route_block(lb, rw_ref.at[pl.ds(SUB_M, SUB_M), :],
                      sel_ref.at[pl.ds(SUB_M, SUB_M), :])

    @pl.when(i == 0)
    def _():
        acc_ref[...] = jnp.zeros_like(acc_ref)

    acc_ref[...] += pa + pb

    @pl.when(i == nblocks - 1)
    def _():
        mean_pe = acc_ref[...] * inv_total
        aux_ref[...] = E * jnp.sum(mean_pe * mean_pe, axis=(0, 1),
                                   keepdims=True)


def kernel(hidden_states, gate_weight):
    b, s, h = hidden_states.shape
    n = b * s
    x = hidden_states.reshape(n, h)
    wt = gate_weight.T
    nblocks = n // BLOCK_M

    body = functools.partial(_router_kernel, nblocks=nblocks,
                             inv_total=1.0 / n)
    rw, sel, aux = pl.pallas_call(
        body,
        grid=(nblocks,),
        in_specs=[
            pl.BlockSpec((SUB_M, h), lambda i: (2 * i, 0)),
            pl.BlockSpec((SUB_M, h), lambda i: (2 * i + 1, 0)),
            pl.BlockSpec((h, E), lambda i: (0, 0)),
        ],
        out_specs=[
            pl.BlockSpec((BLOCK_M, TOPK), lambda i: (i, 0)),
            pl.BlockSpec((BLOCK_M, TOPK), lambda i: (i, 0)),
            pl.BlockSpec((1, 1), lambda i: (0, 0)),
        ],
        out_shape=[
            jax.ShapeDtypeStruct((n, TOPK), jnp.float32),
            jax.ShapeDtypeStruct((n, TOPK), jnp.int32),
            jax.ShapeDtypeStruct((1, 1), jnp.float32),
        ],
        scratch_shapes=[pltpu.VMEM((1, E), jnp.float32)],
    )(x, x, wt)

    return (rw.reshape(b, s, TOPK), sel.reshape(b, s, TOPK), aux[0, 0])


# speedup vs baseline: 1.2006x; 1.0453x over previous
import functools

import jax
import jax.numpy as jnp
from jax.experimental import pallas as pl
from jax.experimental.pallas import tpu as pltpu

TOPK = 2
E = 16
BLOCK_M = 2048
NSTREAM = 2
SUB_M = BLOCK_M // NSTREAM


def _route_block(logits, rw_ref, sel_ref):
    m1 = jnp.max(logits, axis=1, keepdims=True)
    i1 = jnp.argmax(logits, axis=1)
    eidx = jax.lax.broadcasted_iota(jnp.int32, logits.shape, 1)
    masked = jnp.where(eidx == i1[:, None], -jnp.inf, logits)
    m2 = jnp.max(masked, axis=1, keepdims=True)
    i2 = jnp.argmax(masked, axis=1)

    e2 = jnp.exp(m2 - m1)
    denom = 1.0 + e2
    rw_ref[...] = jnp.concatenate([1.0 / denom, e2 / denom], axis=1)
    sel_ref[...] = jnp.concatenate([i1[:, None], i2[:, None]], axis=1)

    p = jnp.exp(logits - m1)
    p = p / jnp.sum(p, axis=1, keepdims=True)
    return jnp.sum(p, axis=0, keepdims=True)


def _router_kernel(xa_ref, xb_ref, wt_ref, rw_ref, sel_ref, aux_ref,
                   acc_ref, *, nblocks, inv_total):
    i = pl.program_id(0)
    w = wt_ref[...]
    dn = (((1,), (1,)), ((), ()))
    la = jax.lax.dot_general(xa_ref[...], w, dn,
                             preferred_element_type=jnp.float32)
    lb = jax.lax.dot_general(xb_ref[...], w, dn,
                             preferred_element_type=jnp.float32)
    pa = _route_block(la, rw_ref.at[pl.ds(0, SUB_M), :],
                      sel_ref.at[pl.ds(0, SUB_M), :])
    pb = _route_block(lb, rw_ref.at[pl.ds(SUB_M, SUB_M), :],
                      sel_ref.at[pl.ds(SUB_M, SUB_M), :])

    @pl.when(i == 0)
    def _():
        acc_ref[...] = jnp.zeros_like(acc_ref)

    acc_ref[...] += pa + pb

    @pl.when(i == nblocks - 1)
    def _():
        mean_pe = acc_ref[...] * inv_total
        aux_ref[...] = E * jnp.sum(mean_pe * mean_pe, axis=(0, 1),
                                   keepdims=True)


def kernel(hidden_states, gate_weight):
    b, s, h = hidden_states.shape
    n = b * s
    x = hidden_states.reshape(n, h)
    nblocks = n // BLOCK_M

    body = functools.partial(_router_kernel, nblocks=nblocks,
                             inv_total=1.0 / n)
    rw, sel, aux = pl.pallas_call(
        body,
        grid=(nblocks,),
        in_specs=[
            pl.BlockSpec((SUB_M, h), lambda i: (2 * i, 0)),
            pl.BlockSpec((SUB_M, h), lambda i: (2 * i + 1, 0)),
            pl.BlockSpec((E, h), lambda i: (0, 0)),
        ],
        out_specs=[
            pl.BlockSpec((BLOCK_M, TOPK), lambda i: (i, 0)),
            pl.BlockSpec((BLOCK_M, TOPK), lambda i: (i, 0)),
            pl.BlockSpec((1, 1), lambda i: (0, 0)),
        ],
        out_shape=[
            jax.ShapeDtypeStruct((n, TOPK), jnp.float32),
            jax.ShapeDtypeStruct((n, TOPK), jnp.int32),
            jax.ShapeDtypeStruct((1, 1), jnp.float32),
        ],
        scratch_shapes=[pltpu.VMEM((1, E), jnp.float32)],
    )(x, x, gate_weight)

    return (rw.reshape(b, s, TOPK), sel.reshape(b, s, TOPK), aux[0, 0])


# speedup vs baseline: 1.2023x; 1.0013x over previous
import functools

import jax
import jax.numpy as jnp
from jax.experimental import pallas as pl
from jax.experimental.pallas import tpu as pltpu

TOPK = 2
E = 16
BLOCK_S = 2048
NSTREAM = 2
SUB_S = BLOCK_S // NSTREAM


def _route_block(logits, rw_ref, sel_ref):
    m1 = jnp.max(logits, axis=1, keepdims=True)
    i1 = jnp.argmax(logits, axis=1)
    eidx = jax.lax.broadcasted_iota(jnp.int32, logits.shape, 1)
    masked = jnp.where(eidx == i1[:, None], -jnp.inf, logits)
    m2 = jnp.max(masked, axis=1, keepdims=True)
    i2 = jnp.argmax(masked, axis=1)

    e2 = jnp.exp(m2 - m1)
    denom = 1.0 + e2
    rw_ref[...] = jnp.concatenate([1.0 / denom, e2 / denom], axis=1)
    sel_ref[...] = jnp.concatenate([i1[:, None], i2[:, None]], axis=1)

    p = jnp.exp(logits - m1)
    p = p / jnp.sum(p, axis=1, keepdims=True)
    return jnp.sum(p, axis=0, keepdims=True)


def _router_kernel(xa_ref, xb_ref, wt_ref, rw_ref, sel_ref, aux_ref,
                   acc_ref, *, nb, nsb, inv_total):
    bi = pl.program_id(0)
    si = pl.program_id(1)
    w = wt_ref[...]
    dn = (((1,), (1,)), ((), ()))
    la = jax.lax.dot_general(xa_ref[0], w, dn,
                             preferred_element_type=jnp.float32)
    lb = jax.lax.dot_general(xb_ref[0], w, dn,
                             preferred_element_type=jnp.float32)
    pa = _route_block(la, rw_ref.at[0, pl.ds(0, SUB_S), :],
                      sel_ref.at[0, pl.ds(0, SUB_S), :])
    pb = _route_block(lb, rw_ref.at[0, pl.ds(SUB_S, SUB_S), :],
                      sel_ref.at[0, pl.ds(SUB_S, SUB_S), :])

    @pl.when((bi == 0) & (si == 0))
    def _():
        acc_ref[...] = jnp.zeros_like(acc_ref)

    acc_ref[...] += pa + pb

    @pl.when((bi == nb - 1) & (si == nsb - 1))
    def _():
        mean_pe = acc_ref[...] * inv_total
        aux_ref[0] = jnp.sum(E * mean_pe * mean_pe)


def kernel(hidden_states, gate_weight):
    b, s, h = hidden_states.shape
    n = b * s
    nsb = s // BLOCK_S

    body = functools.partial(_router_kernel, nb=b, nsb=nsb,
                             inv_total=1.0 / n)
    rw, sel, aux = pl.pallas_call(
        body,
        grid=(b, nsb),
        in_specs=[
            pl.BlockSpec((1, SUB_S, h), lambda bi, si: (bi, 2 * si, 0)),
            pl.BlockSpec((1, SUB_S, h), lambda bi, si: (bi, 2 * si + 1, 0)),
            pl.BlockSpec((E, h), lambda bi, si: (0, 0)),
        ],
        out_specs=[
            pl.BlockSpec((1, BLOCK_S, TOPK), lambda bi, si: (bi, si, 0)),
            pl.BlockSpec((1, BLOCK_S, TOPK), lambda bi, si: (bi, si, 0)),
            pl.BlockSpec(memory_space=pltpu.SMEM),
        ],
        out_shape=[
            jax.ShapeDtypeStruct((b, s, TOPK), jnp.float32),
            jax.ShapeDtypeStruct((b, s, TOPK), jnp.int32),
            jax.ShapeDtypeStruct((1,), jnp.float32),
        ],
        scratch_shapes=[pltpu.VMEM((1, E), jnp.float32)],
    )(hidden_states, hidden_states, gate_weight)

    return (rw, sel, aux[0])


# speedup vs baseline: 1.5246x; 1.2681x over previous
import functools

import jax
import jax.numpy as jnp
from jax.experimental import pallas as pl
from jax.experimental.pallas import tpu as pltpu

TOPK = 2
E = 16
BLOCK_S = 2048
NSTREAM = 2
SUB_S = BLOCK_S // NSTREAM


def _route_block(logits):
    lt = jnp.transpose(logits)
    m1 = jnp.max(lt, axis=0, keepdims=True)
    i1 = jnp.argmax(lt, axis=0).reshape(1, -1)
    eidx = jax.lax.broadcasted_iota(jnp.int32, lt.shape, 0)
    masked = jnp.where(eidx == i1, -jnp.inf, lt)
    m2 = jnp.max(masked, axis=0, keepdims=True)
    i2 = jnp.argmax(masked, axis=0).reshape(1, -1)

    e2 = jnp.exp(m2 - m1)
    denom = 1.0 + e2
    w1 = 1.0 / denom
    w2 = e2 / denom

    p = jnp.exp(lt - m1)
    p = p / jnp.sum(p, axis=0, keepdims=True)
    psum = jnp.sum(p, axis=1, keepdims=True)
    return w1, w2, i1, i2, psum


def _router_kernel(xa_ref, xb_ref, wt_ref, w1_ref, w2_ref, s1_ref, s2_ref,
                   aux_ref, acc_ref, *, nb, nsb, inv_total):
    bi = pl.program_id(0)
    si = pl.program_id(1)
    w = wt_ref[...]
    dn = (((1,), (1,)), ((), ()))
    la = jax.lax.dot_general(xa_ref[0], w, dn,
                             preferred_element_type=jnp.float32)
    lb = jax.lax.dot_general(xb_ref[0], w, dn,
                             preferred_element_type=jnp.float32)
    w1a, w2a, s1a, s2a, pa = _route_block(la)
    w1b, w2b, s1b, s2b, pb = _route_block(lb)
    w1_ref[0, 0:1, pl.ds(0, SUB_S)] = w1a
    w1_ref[0, 0:1, pl.ds(SUB_S, SUB_S)] = w1b
    w2_ref[0, 0:1, pl.ds(0, SUB_S)] = w2a
    w2_ref[0, 0:1, pl.ds(SUB_S, SUB_S)] = w2b
    s1_ref[0, 0:1, pl.ds(0, SUB_S)] = s1a
    s1_ref[0, 0:1, pl.ds(SUB_S, SUB_S)] = s1b
    s2_ref[0, 0:1, pl.ds(0, SUB_S)] = s2a
    s2_ref[0, 0:1, pl.ds(SUB_S, SUB_S)] = s2b

    @pl.when((bi == 0) & (si == 0))
    def _():
        acc_ref[...] = jnp.zeros_like(acc_ref)

    acc_ref[...] += pa + pb

    @pl.when((bi == nb - 1) & (si == nsb - 1))
    def _():
        mean_pe = acc_ref[...] * inv_total
        aux_ref[0] = jnp.sum(E * mean_pe * mean_pe)


def kernel(hidden_states, gate_weight):
    b, s, h = hidden_states.shape
    n = b * s
    nsb = s // BLOCK_S

    body = functools.partial(_router_kernel, nb=b, nsb=nsb,
                             inv_total=1.0 / n)
    plane = pl.BlockSpec((1, 1, BLOCK_S), lambda bi, si: (bi, 0, si))
    plane_shape_f = jax.ShapeDtypeStruct((b, 1, s), jnp.float32)
    plane_shape_i = jax.ShapeDtypeStruct((b, 1, s), jnp.int32)
    w1, w2, s1, s2, aux = pl.pallas_call(
        body,
        grid=(b, nsb),
        in_specs=[
            pl.BlockSpec((1, SUB_S, h), lambda bi, si: (bi, 2 * si, 0)),
            pl.BlockSpec((1, SUB_S, h), lambda bi, si: (bi, 2 * si + 1, 0)),
            pl.BlockSpec((E, h), lambda bi, si: (0, 0)),
        ],
        out_specs=[
            plane, plane, plane, plane,
            pl.BlockSpec(memory_space=pltpu.SMEM),
        ],
        out_shape=[
            plane_shape_f, plane_shape_f, plane_shape_i, plane_shape_i,
            jax.ShapeDtypeStruct((1,), jnp.float32),
        ],
        scratch_shapes=[pltpu.VMEM((E, 1), jnp.float32)],
    )(hidden_states, hidden_states, gate_weight)

    rw = jnp.stack([w1.reshape(b, s), w2.reshape(b, s)], axis=-1)
    sel = jnp.stack([s1.reshape(b, s), s2.reshape(b, s)], axis=-1)
    return (rw, sel, aux[0])


# speedup vs baseline: 1.5492x; 1.0162x over previous
import functools

import jax
import jax.numpy as jnp
from jax.experimental import pallas as pl
from jax.experimental.pallas import tpu as pltpu

TOPK = 2
E = 16
BLOCK_S = 1024
NSTREAM = 2
SUB_S = BLOCK_S // NSTREAM


def _route_block(logits):
    lt = jnp.transpose(logits)
    m1 = jnp.max(lt, axis=0, keepdims=True)
    i1 = jnp.argmax(lt, axis=0).reshape(1, -1)
    eidx = jax.lax.broadcasted_iota(jnp.int32, lt.shape, 0)
    masked = jnp.where(eidx == i1, -jnp.inf, lt)
    m2 = jnp.max(masked, axis=0, keepdims=True)
    i2 = jnp.argmax(masked, axis=0).reshape(1, -1)

    e2 = jnp.exp(m2 - m1)
    denom = 1.0 + e2
    w1 = 1.0 / denom
    w2 = e2 / denom

    p = jnp.exp(lt - m1)
    p = p / jnp.sum(p, axis=0, keepdims=True)
    psum = jnp.sum(p, axis=1, keepdims=True)
    return w1, w2, i1, i2, psum


def _router_kernel(xa_ref, xb_ref, wt_ref, w1_ref, w2_ref, s1_ref, s2_ref,
                   aux_ref, acc_ref, *, nb, nsb, inv_total):
    bi = pl.program_id(0)
    si = pl.program_id(1)
    w = wt_ref[...]
    dn = (((1,), (1,)), ((), ()))
    la = jax.lax.dot_general(xa_ref[0], w, dn,
                             preferred_element_type=jnp.float32)
    lb = jax.lax.dot_general(xb_ref[0], w, dn,
                             preferred_element_type=jnp.float32)
    w1a, w2a, s1a, s2a, pa = _route_block(la)
    w1b, w2b, s1b, s2b, pb = _route_block(lb)
    w1_ref[0, 0:1, pl.ds(0, SUB_S)] = w1a
    w1_ref[0, 0:1, pl.ds(SUB_S, SUB_S)] = w1b
    w2_ref[0, 0:1, pl.ds(0, SUB_S)] = w2a
    w2_ref[0, 0:1, pl.ds(SUB_S, SUB_S)] = w2b
    s1_ref[0, 0:1, pl.ds(0, SUB_S)] = s1a
    s1_ref[0, 0:1, pl.ds(SUB_S, SUB_S)] = s1b
    s2_ref[0, 0:1, pl.ds(0, SUB_S)] = s2a
    s2_ref[0, 0:1, pl.ds(SUB_S, SUB_S)] = s2b

    @pl.when((bi == 0) & (si == 0))
    def _():
        acc_ref[...] = jnp.zeros_like(acc_ref)

    acc_ref[...] += pa + pb

    @pl.when((bi == nb - 1) & (si == nsb - 1))
    def _():
        mean_pe = acc_ref[...] * inv_total
        aux_ref[0] = jnp.sum(E * mean_pe * mean_pe)


def kernel(hidden_states, gate_weight):
    b, s, h = hidden_states.shape
    n = b * s
    nsb = s // BLOCK_S

    body = functools.partial(_router_kernel, nb=b, nsb=nsb,
                             inv_total=1.0 / n)
    plane = pl.BlockSpec((1, 1, BLOCK_S), lambda bi, si: (bi, 0, si))
    plane_shape_f = jax.ShapeDtypeStruct((b, 1, s), jnp.float32)
    plane_shape_i = jax.ShapeDtypeStruct((b, 1, s), jnp.int32)
    w1, w2, s1, s2, aux = pl.pallas_call(
        body,
        grid=(b, nsb),
        in_specs=[
            pl.BlockSpec((1, SUB_S, h), lambda bi, si: (bi, 2 * si, 0)),
            pl.BlockSpec((1, SUB_S, h), lambda bi, si: (bi, 2 * si + 1, 0)),
            pl.BlockSpec((E, h), lambda bi, si: (0, 0)),
        ],
        out_specs=[
            plane, plane, plane, plane,
            pl.BlockSpec(memory_space=pltpu.SMEM),
        ],
        out_shape=[
            plane_shape_f, plane_shape_f, plane_shape_i, plane_shape_i,
            jax.ShapeDtypeStruct((1,), jnp.float32),
        ],
        scratch_shapes=[pltpu.VMEM((E, 1), jnp.float32)],
    )(hidden_states, hidden_states, gate_weight)

    rw = jnp.stack([w1.reshape(b, s), w2.reshape(b, s)], axis=-1)
    sel = jnp.stack([s1.reshape(b, s), s2.reshape(b, s)], axis=-1)
    return (rw, sel, aux[0])
